# baseline (speedup 1.0000x reference)
import jax
import jax.numpy as jnp
from jax.experimental import pallas as pl
from jax.experimental.pallas import tpu as pltpu

K = 64
S = 4
DIM = 128
ALPHA = 100.0
EPS = 1e-12
BB = 4


def _split_bf16(a):
    hi = a.astype(jnp.bfloat16)
    lo = (a - hi.astype(jnp.float32)).astype(jnp.bfloat16)
    return hi, lo


def _dot3(ah, al, bh, bl, dims):
    d = lambda u, v: jax.lax.dot_general(u, v, dims,
                                         preferred_element_type=jnp.float32)
    return d(ah, bh) + d(ah, bl) + d(al, bh)


def _sral_kernel(xh_ref, xl_ref, cent_ref, out_ref):
    lg2e = 1.4426950408889634
    cf = jnp.concatenate([cent_ref[:, s, :] for s in range(S + 1)],
                         axis=0)
    b = (-ALPHA * lg2e) * jnp.sqrt(jnp.sum(cf * cf, axis=1, keepdims=True))
    wh, wl = _split_bf16((2.0 * ALPHA * lg2e) * cf)
    rep = cf[0:K, :]

    for i in range(BB):
        xh = xh_ref[i]
        xl = xl_ref[i]

        nt = (((1,), (0,)), ((), ()))
        conv = _dot3(wh, wl, xh, xl, nt) + b
        cvs = [conv[s * K:(s + 1) * K, :] for s in range(S + 1)]

        ms = cvs[0]
        for s in range(1, S + 1):
            ms = jnp.maximum(ms, cvs[s])
        ts, sum_a = [], []
        den = None
        for s in range(S + 1):
            mk = jnp.max(cvs[s], axis=0, keepdims=True)
            f = jnp.exp2(cvs[s] - mk)
            sum_a.append(jnp.sum(f, axis=0, keepdims=True))
            e = jnp.exp2(cvs[s] - ms)
            ts.append(f * e)
            den = e if den is None else den + e

        num = sum_a[0] * den + ts[0]
        sum_a_prod = sum_a[0]
        for s in range(1, S + 1):
            num = num * (sum_a[s] * den + ts[s])
            sum_a_prod = sum_a_prod * sum_a[s]
        den2 = den * den
        den5 = den2 * den2 * den
        mult = num / (den5 * sum_a_prod)

        tt = (((1,), (1,)), ((), ()))
        mh = mult.astype(jnp.bfloat16)
        agg = (jax.lax.dot_general(mh, xh, tt, preferred_element_type=jnp.float32)
               + jax.lax.dot_general(mh, xl, tt, preferred_element_type=jnp.float32))
        out = agg - rep * jnp.sum(mult, axis=1, keepdims=True)

        rss = jnp.sum(out * out, axis=1, keepdims=True)
        out = out * jax.lax.rsqrt(jnp.maximum(rss, EPS * EPS))
        gss = jnp.sum(out * out)
        out_ref[i] = out * jax.lax.rsqrt(jnp.maximum(gss, EPS * EPS))


@jax.jit
def kernel(x, centroids):
    N, C, H, W = x.shape
    P = H * W
    xf = x.reshape(N, C, P)
    xh, xl = _split_bf16(xf)

    out = pl.pallas_call(
        _sral_kernel,
        grid=(N // BB,),
        in_specs=[
            pl.BlockSpec((BB, C, P), lambda n: (n, 0, 0)),
            pl.BlockSpec((BB, C, P), lambda n: (n, 0, 0)),
            pl.BlockSpec((K, S + 1, DIM), lambda n: (0, 0, 0)),
        ],
        out_specs=pl.BlockSpec((BB, K, DIM), lambda n: (n, 0, 0)),
        out_shape=jax.ShapeDtypeStruct((N, K, DIM), jnp.float32),
        compiler_params=pltpu.CompilerParams(
            dimension_semantics=("parallel",),
        ),
    )(xh, xl, centroids)
    return out.reshape(N, K * DIM)


# speedup vs baseline: 1.6722x; 1.6722x over previous
import jax
import jax.numpy as jnp
from jax.experimental import pallas as pl
from jax.experimental.pallas import tpu as pltpu

K = 64
S = 4
DIM = 128
ALPHA = 100.0
EPS = 1e-12
BB = 4


def _split_bf16(a):
    hi = a.astype(jnp.bfloat16)
    lo = (a - hi.astype(jnp.float32)).astype(jnp.bfloat16)
    return hi, lo


def _dot3(ah, al, bh, bl, dims):
    d = lambda u, v: jax.lax.dot_general(u, v, dims,
                                         preferred_element_type=jnp.float32)
    return d(ah, bh) + d(ah, bl) + d(al, bh)


def _sral_kernel(x_ref, cent_ref, out_ref):
    lg2e = 1.4426950408889634
    cf = jnp.concatenate([cent_ref[:, s, :] for s in range(S + 1)],
                         axis=0)
    b = (-ALPHA * lg2e) * jnp.sqrt(jnp.sum(cf * cf, axis=1, keepdims=True))
    wh, wl = _split_bf16((2.0 * ALPHA * lg2e) * cf)
    rep = cf[0:K, :]

    for i in range(BB):
        xf = x_ref[i]
        xh, xl = _split_bf16(xf)

        nt = (((1,), (0,)), ((), ()))
        conv = _dot3(wh, wl, xh, xl, nt) + b
        cvs = [conv[s * K:(s + 1) * K, :] for s in range(S + 1)]

        ms = cvs[0]
        for s in range(1, S + 1):
            ms = jnp.maximum(ms, cvs[s])
        ts, sum_a = [], []
        den = None
        for s in range(S + 1):
            mk = jnp.max(cvs[s], axis=0, keepdims=True)
            f = jnp.exp2(cvs[s] - mk)
            sum_a.append(jnp.sum(f, axis=0, keepdims=True))
            e = jnp.exp2(cvs[s] - ms)
            ts.append(f * e)
            den = e if den is None else den + e

        num = sum_a[0] * den + ts[0]
        sum_a_prod = sum_a[0]
        for s in range(1, S + 1):
            num = num * (sum_a[s] * den + ts[s])
            sum_a_prod = sum_a_prod * sum_a[s]
        den2 = den * den
        den5 = den2 * den2 * den
        mult = num / (den5 * sum_a_prod)

        tt = (((1,), (1,)), ((), ()))
        mh = mult.astype(jnp.bfloat16)
        agg = (jax.lax.dot_general(mh, xh, tt, preferred_element_type=jnp.float32)
               + jax.lax.dot_general(mh, xl, tt, preferred_element_type=jnp.float32))
        out = agg - rep * jnp.sum(mult, axis=1, keepdims=True)

        rss = jnp.sum(out * out, axis=1, keepdims=True)
        out = out * jax.lax.rsqrt(jnp.maximum(rss, EPS * EPS))
        gss = jnp.sum(out * out)
        out_ref[i] = out * jax.lax.rsqrt(jnp.maximum(gss, EPS * EPS))


@jax.jit
def kernel(x, centroids):
    N, C, H, W = x.shape
    P = H * W
    xf = x.reshape(N, C, P)

    out = pl.pallas_call(
        _sral_kernel,
        grid=(N // BB,),
        in_specs=[
            pl.BlockSpec((BB, C, P), lambda n: (n, 0, 0)),
            pl.BlockSpec((K, S + 1, DIM), lambda n: (0, 0, 0)),
        ],
        out_specs=pl.BlockSpec((BB, K, DIM), lambda n: (n, 0, 0)),
        out_shape=jax.ShapeDtypeStruct((N, K, DIM), jnp.float32),
        compiler_params=pltpu.CompilerParams(
            dimension_semantics=("parallel",),
        ),
    )(xf, centroids)
    return out.reshape(N, K * DIM)


# speedup vs baseline: 1.7497x; 1.0463x over previous
import jax
import jax.numpy as jnp
from jax.experimental import pallas as pl
from jax.experimental.pallas import tpu as pltpu

K = 64
S = 4
DIM = 128
ALPHA = 100.0
EPS = 1e-12
BB = 4


def _split_bf16(a):
    hi = a.astype(jnp.bfloat16)
    lo = (a - hi.astype(jnp.float32)).astype(jnp.bfloat16)
    return hi, lo


def _dot3(ah, al, bh, bl, dims):
    d = lambda u, v: jax.lax.dot_general(u, v, dims,
                                         preferred_element_type=jnp.float32)
    return d(ah, bh) + d(ah, bl) + d(al, bh)


def _sral_kernel(x_ref, cent_ref, out_ref):
    lg2e = 1.4426950408889634
    cf = jnp.concatenate([cent_ref[:, s, :] for s in range(S + 1)],
                         axis=0)
    b = (-ALPHA * lg2e) * jnp.sqrt(jnp.sum(cf * cf, axis=1, keepdims=True))
    wh, wl = _split_bf16((2.0 * ALPHA * lg2e) * cf)
    wa = jnp.concatenate([wh, wl], axis=1)
    wa2 = jnp.concatenate([wl, wh], axis=1)
    rep = cf[0:K, :]

    for i in range(BB):
        xf = x_ref[i]
        xh, xl = _split_bf16(xf)

        nt = (((1,), (0,)), ((), ()))
        xb = jnp.concatenate([xh, xl], axis=0)
        conv = (jax.lax.dot_general(wa, xb, nt,
                                    preferred_element_type=jnp.float32)
                + jax.lax.dot_general(wa2, xb, nt,
                                      preferred_element_type=jnp.float32)
                + b)
        cvs = [conv[s * K:(s + 1) * K, :] for s in range(S + 1)]

        ms = cvs[0]
        for s in range(1, S + 1):
            ms = jnp.maximum(ms, cvs[s])
        ts, sum_a = [], []
        den = None
        for s in range(S + 1):
            mk = jnp.max(cvs[s], axis=0, keepdims=True)
            f = jnp.exp2(cvs[s] - mk)
            sum_a.append(jnp.sum(f, axis=0, keepdims=True))
            e = jnp.exp2(cvs[s] - ms)
            ts.append(f * e)
            den = e if den is None else den + e

        num = sum_a[0] * den + ts[0]
        sum_a_prod = sum_a[0]
        for s in range(1, S + 1):
            num = num * (sum_a[s] * den + ts[s])
            sum_a_prod = sum_a_prod * sum_a[s]
        den2 = den * den
        den5 = den2 * den2 * den
        mult = num / (den5 * sum_a_prod)

        tt = (((1,), (1,)), ((), ()))
        mh = mult.astype(jnp.bfloat16)
        agg = (jax.lax.dot_general(mh, xh, tt, preferred_element_type=jnp.float32)
               + jax.lax.dot_general(mh, xl, tt, preferred_element_type=jnp.float32))
        out = agg - rep * jnp.sum(mult, axis=1, keepdims=True)

        rss = jnp.sum(out * out, axis=1, keepdims=True)
        out = out * jax.lax.rsqrt(jnp.maximum(rss, EPS * EPS))
        gss = jnp.sum(out * out)
        out_ref[i] = out * jax.lax.rsqrt(jnp.maximum(gss, EPS * EPS))


@jax.jit
def kernel(x, centroids):
    N, C, H, W = x.shape
    P = H * W
    xf = x.reshape(N, C, P)

    out = pl.pallas_call(
        _sral_kernel,
        grid=(N // BB,),
        in_specs=[
            pl.BlockSpec((BB, C, P), lambda n: (n, 0, 0)),
            pl.BlockSpec((K, S + 1, DIM), lambda n: (0, 0, 0)),
        ],
        out_specs=pl.BlockSpec((BB, K, DIM), lambda n: (n, 0, 0)),
        out_shape=jax.ShapeDtypeStruct((N, K, DIM), jnp.float32),
        compiler_params=pltpu.CompilerParams(
            dimension_semantics=("parallel",),
        ),
    )(xf, centroids)
    return out.reshape(N, K * DIM)


# speedup vs baseline: 1.7942x; 1.0255x over previous
import jax
import jax.numpy as jnp
from jax.experimental import pallas as pl
from jax.experimental.pallas import tpu as pltpu

K = 64
S = 4
DIM = 128
ALPHA = 100.0
EPS = 1e-12
BB = 4


def _split_bf16(a):
    hi = a.astype(jnp.bfloat16)
    lo = (a - hi.astype(jnp.float32)).astype(jnp.bfloat16)
    return hi, lo


def _dot3(ah, al, bh, bl, dims):
    d = lambda u, v: jax.lax.dot_general(u, v, dims,
                                         preferred_element_type=jnp.float32)
    return d(ah, bh) + d(ah, bl) + d(al, bh)


def _sral_kernel(x_ref, cent_ref, out_ref):
    lg2e = 1.4426950408889634
    cf = jnp.concatenate([cent_ref[:, s, :] for s in range(S + 1)],
                         axis=0)
    b = (-ALPHA * lg2e) * jnp.sqrt(jnp.sum(cf * cf, axis=1, keepdims=True))
    wh, wl = _split_bf16((2.0 * ALPHA * lg2e) * cf)
    wa = jnp.concatenate([wh, wl], axis=1)
    wa2 = jnp.concatenate([wl, wh], axis=1)
    rep = cf[0:K, :]

    for i in range(BB):
        xf = x_ref[i]
        xh, xl = _split_bf16(xf)

        nt = (((1,), (0,)), ((), ()))
        xb = jnp.concatenate([xh, xl], axis=0)
        conv = (jax.lax.dot_general(wa, xb, nt,
                                    preferred_element_type=jnp.float32)
                + jax.lax.dot_general(wa2, xb, nt,
                                      preferred_element_type=jnp.float32)
                + b)
        cvs = [conv[s * K:(s + 1) * K, :] for s in range(S + 1)]

        ms = cvs[0]
        for s in range(1, S + 1):
            ms = jnp.maximum(ms, cvs[s])
        ts, sum_a = [], []
        den = None
        for s in range(S + 1):
            mk = jnp.max(cvs[s], axis=0, keepdims=True)
            f = jnp.exp2(cvs[s] - mk)
            sum_a.append(jnp.sum(f, axis=0, keepdims=True))
            e = jnp.exp2(cvs[s] - ms)
            ts.append(f * e)
            den = e if den is None else den + e

        num = sum_a[0] * den + ts[0]
        sum_a_prod = sum_a[0]
        for s in range(1, S + 1):
            num = num * (sum_a[s] * den + ts[s])
            sum_a_prod = sum_a_prod * sum_a[s]
        den2 = den * den
        den5 = den2 * den2 * den
        mult = num / (den5 * sum_a_prod)

        tt = (((1,), (1,)), ((), ()))
        mh = mult.astype(jnp.bfloat16)
        agg2 = jax.lax.dot_general(mh, xb, tt,
                                   preferred_element_type=jnp.float32)
        agg = agg2[:, 0:DIM] + agg2[:, DIM:2 * DIM]
        out = agg - rep * jnp.sum(mult, axis=1, keepdims=True)

        rss = jnp.sum(out * out, axis=1, keepdims=True)
        out = out * jax.lax.rsqrt(jnp.maximum(rss, EPS * EPS))
        gss = jnp.sum(out * out)
        out_ref[i] = out * jax.lax.rsqrt(jnp.maximum(gss, EPS * EPS))


@jax.jit
def kernel(x, centroids):
    N, C, H, W = x.shape
    P = H * W
    xf = x.reshape(N, C, P)

    out = pl.pallas_call(
        _sral_kernel,
        grid=(N // BB,),
        in_specs=[
            pl.BlockSpec((BB, C, P), lambda n: (n, 0, 0)),
            pl.BlockSpec((K, S + 1, DIM), lambda n: (0, 0, 0)),
        ],
        out_specs=pl.BlockSpec((BB, K, DIM), lambda n: (n, 0, 0)),
        out_shape=jax.ShapeDtypeStruct((N, K, DIM), jnp.float32),
        compiler_params=pltpu.CompilerParams(
            dimension_semantics=("parallel",),
        ),
    )(xf, centroids)
    return out.reshape(N, K * DIM)
